# Initial kernel scaffold; baseline (speedup 1.0000x reference)
import jax
import jax.numpy as jnp
from jax.experimental import pallas as pl


def kernel(features, edge_index, W1, b1, W2, b2):
    raise NotImplementedError("write your pallas kernel here")


# speedup vs baseline: 1.2008x; 1.2008x over previous
import functools

import jax
import jax.numpy as jnp
from jax import lax
from jax.experimental import pallas as pl
from jax.experimental.pallas import tpu as pltpu
from jax.experimental.pallas import tpu_sc as plsc

N = 10000
E = 320000
D = 128

NC = 2
NS = 16
NW = NC * NS
EPW = E // NW
K = 80
ITERS = EPW // K
NWB = 10
RPW = N // NWB

_MESH = plsc.VectorSubcoreMesh(core_axis_name="c", subcore_axis_name="s")


@functools.partial(
    pl.kernel,
    out_type=(
        jax.ShapeDtypeStruct((NC, N, 16), jnp.float32),
        jax.ShapeDtypeStruct((NC, N, 16), jnp.float32),
    ),
    mesh=_MESH,
    scratch_types=[
        pltpu.VMEM((K,), jnp.int32),
        pltpu.VMEM((K,), jnp.int32),
        pltpu.VMEM((K, 16), jnp.float32),
        pltpu.VMEM_SHARED((N, 16), jnp.float32),
        pltpu.VMEM_SHARED((N, 16), jnp.float32),
    ],
)
def _deg_kernel(src_hbm, dst_hbm, ones_hbm, zeros_hbm,
                od_out, id_out, sidx, didx, ones_v, od_tab, id_tab):
    c = lax.axis_index("c")
    s = lax.axis_index("s")
    base = (c * NS + s) * EPW
    pltpu.sync_copy(ones_hbm, ones_v)
    @pl.when(s < NWB)
    def _zero():
        pltpu.sync_copy(zeros_hbm, od_tab.at[pl.ds(s * RPW, RPW)])
        pltpu.sync_copy(zeros_hbm, id_tab.at[pl.ds(s * RPW, RPW)])
    plsc.subcore_barrier()

    def body(j, _):
        pltpu.sync_copy(src_hbm.at[pl.ds(base + j * K, K)], sidx)
        pltpu.sync_copy(dst_hbm.at[pl.ds(base + j * K, K)], didx)
        pltpu.sync_copy(ones_v, od_tab.at[sidx], add=True)
        pltpu.sync_copy(ones_v, id_tab.at[didx], add=True)
        return ()

    lax.fori_loop(0, ITERS, body, ())
    plsc.subcore_barrier()

    @pl.when(s < NWB)
    def _writeback():
        pltpu.sync_copy(od_tab.at[pl.ds(s * RPW, RPW)], od_out.at[c, pl.ds(s * RPW, RPW)])
        pltpu.sync_copy(id_tab.at[pl.ds(s * RPW, RPW)], id_out.at[c, pl.ds(s * RPW, RPW)])


@functools.partial(
    pl.kernel,
    out_type=jax.ShapeDtypeStruct((NC, N, D), jnp.float32),
    mesh=_MESH,
    scratch_types=[
        pltpu.VMEM((ITERS, K), jnp.int32),
        pltpu.VMEM((ITERS, K), jnp.int32),
        pltpu.VMEM((K, D), jnp.float32),
        pltpu.VMEM_SHARED((N, D), jnp.float32),
        pltpu.SemaphoreType.DMA,
    ],
)
def _agg_kernel(h_hbm, src_hbm, dst_hbm, zeros_hbm,
                out_hbm, sidx, didx, rows, acc, sem):
    c = lax.axis_index("c")
    s = lax.axis_index("s")
    wid = c * NS + s
    pltpu.sync_copy(src_hbm.at[wid], sidx)
    pltpu.sync_copy(dst_hbm.at[wid], didx)
    @pl.when(s < NWB)
    def _zero():
        pltpu.sync_copy(zeros_hbm, acc.at[pl.ds(s * RPW, RPW)])
    plsc.subcore_barrier()

    def body(j, _):
        pltpu.async_copy(h_hbm.at[sidx.at[j]], rows, sem).wait()
        pltpu.sync_copy(rows, acc.at[didx.at[j]], add=True)
        return ()

    lax.fori_loop(0, ITERS, body, ())
    plsc.subcore_barrier()

    @pl.when(s < NWB)
    def _writeback():
        pltpu.sync_copy(acc.at[pl.ds(s * RPW, RPW)], out_hbm.at[c, pl.ds(s * RPW, RPW)])


_ROWS = 1000


def _prep_body(f_ref, od_ref, o_ref):
    deg = od_ref[0, :, 0:1] + od_ref[1, :, 0:1]
    norm = lax.rsqrt(jnp.maximum(deg, 1.0))
    o_ref[...] = f_ref[...] * norm


def _tc_prep(features, od):
    return pl.pallas_call(
        _prep_body,
        out_shape=jax.ShapeDtypeStruct((N, D), jnp.float32),
        grid=(N // _ROWS,),
        in_specs=[
            pl.BlockSpec((_ROWS, D), lambda i: (i, 0)),
            pl.BlockSpec((NC, _ROWS, 16), lambda i: (0, i, 0)),
        ],
        out_specs=pl.BlockSpec((_ROWS, D), lambda i: (i, 0)),
    )(features, od)


def _dense_body(apply_src_norm, agg_ref, id_ref, w_ref, b_ref, od_ref, o_ref):
    agg = agg_ref[0] + agg_ref[1]
    ideg = id_ref[0, :, 0:1] + id_ref[1, :, 0:1]
    nd = lax.rsqrt(jnp.maximum(ideg, 1.0))
    x = agg * nd
    y = jnp.dot(x, w_ref[...], preferred_element_type=jnp.float32) + b_ref[...]
    y = jnp.maximum(y, 0.0)
    if apply_src_norm:
        odeg = od_ref[0, :, 0:1] + od_ref[1, :, 0:1]
        y = y * lax.rsqrt(jnp.maximum(odeg, 1.0))
    o_ref[...] = y


def _tc_dense(aggP, idslab, W, b, odslab, apply_src_norm):
    return pl.pallas_call(
        functools.partial(_dense_body, apply_src_norm),
        out_shape=jax.ShapeDtypeStruct((N, D), jnp.float32),
        grid=(N // _ROWS,),
        in_specs=[
            pl.BlockSpec((NC, _ROWS, D), lambda i: (0, i, 0)),
            pl.BlockSpec((NC, _ROWS, 16), lambda i: (0, i, 0)),
            pl.BlockSpec((D, D), lambda i: (0, 0)),
            pl.BlockSpec((1, D), lambda i: (0, 0)),
            pl.BlockSpec((NC, _ROWS, 16), lambda i: (0, i, 0)),
        ],
        out_specs=pl.BlockSpec((_ROWS, D), lambda i: (i, 0)),
    )(aggP, idslab, W, b, odslab)


def kernel(features, edge_index, W1, b1, W2, b2):
    src = edge_index[0]
    dst = edge_index[1]
    ones16 = jnp.ones((K, 16), jnp.float32)
    zeros16 = jnp.zeros((RPW, 16), jnp.float32)
    zerosD = jnp.zeros((RPW, D), jnp.float32)

    src3 = src.reshape(NW, ITERS, K)
    dst3 = dst.reshape(NW, ITERS, K)
    od, idg = _deg_kernel(src, dst, ones16, zeros16)
    h0 = _tc_prep(features, od)

    def _jagg(h):
        msg = jnp.take(h, src, axis=0)
        a = jnp.zeros((N, D), h.dtype).at[dst].add(msg)
        return jnp.stack([a, jnp.zeros_like(a)])

    agg1 = _jagg(h0)
    h1 = _tc_dense(agg1, idg, W1, b1.reshape(1, D), od, True)
    agg2 = _jagg(h1)
    out = _tc_dense(agg2, idg, W2, b2.reshape(1, D), od, False)
    return out


# speedup vs baseline: 2.4532x; 2.0430x over previous
import functools

import jax
import jax.numpy as jnp
from jax import lax
from jax.experimental import pallas as pl
from jax.experimental.pallas import tpu as pltpu
from jax.experimental.pallas import tpu_sc as plsc

N = 10000
NPAD = 10240
E = 320000
D = 128

NC = 2
NS = 16
NW = NC * NS
EPW = E // NW
CPT = D // NW
CH = 2000
NCH = E // CH

_MESH = plsc.VectorSubcoreMesh(core_axis_name="c", subcore_axis_name="s")
_SC_PARAMS = pltpu.CompilerParams(needs_layout_passes=False)


@functools.partial(
    pl.kernel,
    out_type=jax.ShapeDtypeStruct((NW, 2, NPAD), jnp.float32),
    mesh=_MESH,
    compiler_params=_SC_PARAMS,
    scratch_types=[
        pltpu.VMEM((EPW,), jnp.int32),
        pltpu.VMEM((EPW,), jnp.int32),
        pltpu.VMEM((2, NPAD), jnp.float32),
    ],
)
def _deg_kernel(src_hbm, dst_hbm, out_hbm, sidx, didx, odid):
    c = lax.axis_index("c")
    s = lax.axis_index("s")
    wid = c * NS + s
    base = wid * EPW
    pltpu.sync_copy(src_hbm.at[pl.ds(base, EPW)], sidx)
    pltpu.sync_copy(dst_hbm.at[pl.ds(base, EPW)], didx)

    zero16 = jnp.zeros((16,), jnp.float32)

    def zbody(i, _):
        odid[0, pl.ds(i * 16, 16)] = zero16
        odid[1, pl.ds(i * 16, 16)] = zero16
        return ()

    lax.fori_loop(0, NPAD // 16, zbody, ())

    ones16 = jnp.ones((16,), jnp.float32)
    row0 = jnp.zeros((16,), jnp.int32)
    row1 = jnp.ones((16,), jnp.int32)

    def body(i, _):
        s16 = sidx[pl.ds(i * 16, 16)]
        d16 = didx[pl.ds(i * 16, 16)]
        plsc.addupdate_scatter(odid, [row0, s16], ones16)
        plsc.addupdate_scatter(odid, [row1, d16], ones16)
        return ()

    lax.fori_loop(0, EPW // 16, body, ())
    pltpu.sync_copy(odid, out_hbm.at[wid])


@functools.partial(
    pl.kernel,
    out_type=jax.ShapeDtypeStruct((NW, CPT, NPAD), jnp.float32),
    mesh=_MESH,
    compiler_params=_SC_PARAMS,
    scratch_types=[
        pltpu.VMEM((CH,), jnp.int32),
        pltpu.VMEM((CH,), jnp.int32),
        pltpu.VMEM((CPT, NPAD), jnp.float32),
        pltpu.VMEM((CPT, NPAD), jnp.float32),
    ],
)
def _agg_kernel(ht_hbm, src_hbm, dst_hbm, out_hbm, sidx, didx, tab, acc):
    c = lax.axis_index("c")
    s = lax.axis_index("s")
    wid = c * NS + s
    pltpu.sync_copy(ht_hbm.at[wid], tab)

    zero16 = jnp.zeros((16,), jnp.float32)

    def zbody(i, _):
        for j in range(CPT):
            acc[j, pl.ds(i * 16, 16)] = zero16
        return ()

    lax.fori_loop(0, NPAD // 16, zbody, ())

    jrows = [jnp.full((16,), j, jnp.int32) for j in range(CPT)]

    def chunk_body(k, _):
        pltpu.sync_copy(src_hbm.at[pl.ds(k * CH, CH)], sidx)
        pltpu.sync_copy(dst_hbm.at[pl.ds(k * CH, CH)], didx)

        def body(i, _):
            s16 = sidx[pl.ds(i * 16, 16)]
            d16 = didx[pl.ds(i * 16, 16)]
            for j in range(CPT):
                vals = plsc.load_gather(tab, [jrows[j], s16])
                plsc.addupdate_scatter(acc, [jrows[j], d16], vals)
            return ()

        lax.fori_loop(0, CH // 16, body, ())
        return ()

    lax.fori_loop(0, NCH, chunk_body, ())
    pltpu.sync_copy(acc, out_hbm.at[wid])


_R = 1024


def _prep_body(ft_ref, deg_ref, o_ref):
    deg = jnp.sum(deg_ref[...], axis=0)
    ns = lax.rsqrt(jnp.maximum(deg[0:1, :], 1.0))
    o_ref[...] = ft_ref[...] * ns


def _tc_prep(ft, degP):
    return pl.pallas_call(
        _prep_body,
        out_shape=jax.ShapeDtypeStruct((D, NPAD), jnp.float32),
        grid=(NPAD // _R,),
        in_specs=[
            pl.BlockSpec((D, _R), lambda i: (0, i)),
            pl.BlockSpec((NW, 2, _R), lambda i: (0, 0, i)),
        ],
        out_specs=pl.BlockSpec((D, _R), lambda i: (0, i)),
    )(ft, degP)


def _dense_body(apply_src_norm, aggt_ref, deg_ref, w_ref, b_ref, o_ref):
    deg = jnp.sum(deg_ref[...], axis=0)
    nd = lax.rsqrt(jnp.maximum(deg[1:2, :], 1.0))
    x = aggt_ref[...] * nd
    y = lax.dot_general(w_ref[...], x, (((0,), (0,)), ((), ())),
                        preferred_element_type=jnp.float32)
    y = y + b_ref[:, 0:1]
    y = jnp.maximum(y, 0.0)
    if apply_src_norm:
        ns = lax.rsqrt(jnp.maximum(deg[0:1, :], 1.0))
        y = y * ns
    o_ref[...] = y


def _tc_dense(aggT, degP, W, b_bc, apply_src_norm):
    return pl.pallas_call(
        functools.partial(_dense_body, apply_src_norm),
        out_shape=jax.ShapeDtypeStruct((D, NPAD), jnp.float32),
        grid=(NPAD // _R,),
        in_specs=[
            pl.BlockSpec((D, _R), lambda i: (0, i)),
            pl.BlockSpec((NW, 2, _R), lambda i: (0, 0, i)),
            pl.BlockSpec((D, D), lambda i: (0, 0)),
            pl.BlockSpec((D, D), lambda i: (0, 0)),
        ],
        out_specs=pl.BlockSpec((D, _R), lambda i: (0, i)),
    )(aggT, degP, W, b_bc)


def kernel(features, edge_index, W1, b1, W2, b2):
    src = edge_index[0]
    dst = edge_index[1]

    degP = _deg_kernel(src, dst)

    ft = jnp.zeros((D, NPAD), jnp.float32).at[:, :N].set(features.T)
    hT0 = _tc_prep(ft, degP)

    b1_bc = jnp.broadcast_to(b1.reshape(D, 1), (D, D))
    b2_bc = jnp.broadcast_to(b2.reshape(D, 1), (D, D))

    agg1 = _agg_kernel(hT0.reshape(NW, CPT, NPAD), src, dst)
    h1T = _tc_dense(agg1.reshape(D, NPAD), degP, W1, b1_bc, True)

    agg2 = _agg_kernel(h1T.reshape(NW, CPT, NPAD), src, dst)
    o2T = _tc_dense(agg2.reshape(D, NPAD), degP, W2, b2_bc, False)
    return o2T[:, :N].T


# speedup vs baseline: 3.3159x; 1.3517x over previous
import functools

import jax
import jax.numpy as jnp
from jax import lax
from jax.experimental import pallas as pl
from jax.experimental.pallas import tpu as pltpu
from jax.experimental.pallas import tpu_sc as plsc

N = 10000
NPAD = 10240
E = 320000
D = 128

NC = 2
NS = 16
NW = NC * NS
EPW = E // NW
CPT = D // NW
CH = 6400
NCH = E // CH
UNROLL = 4

_MESH = plsc.VectorSubcoreMesh(core_axis_name="c", subcore_axis_name="s")
_SC_PARAMS = pltpu.CompilerParams(needs_layout_passes=False)


@functools.partial(
    pl.kernel,
    out_type=jax.ShapeDtypeStruct((NW, 2, NPAD), jnp.float32),
    mesh=_MESH,
    compiler_params=_SC_PARAMS,
    scratch_types=[
        pltpu.VMEM((EPW,), jnp.int32),
        pltpu.VMEM((EPW,), jnp.int32),
        pltpu.VMEM((2, NPAD), jnp.float32),
    ],
)
def _deg_kernel(src_hbm, dst_hbm, out_hbm, sidx, didx, odid):
    c = lax.axis_index("c")
    s = lax.axis_index("s")
    wid = c * NS + s
    base = wid * EPW
    pltpu.sync_copy(src_hbm.at[pl.ds(base, EPW)], sidx)
    pltpu.sync_copy(dst_hbm.at[pl.ds(base, EPW)], didx)

    zero16 = jnp.zeros((16,), jnp.float32)

    def zbody(i, _):
        odid[0, pl.ds(i * 16, 16)] = zero16
        odid[1, pl.ds(i * 16, 16)] = zero16
        return ()

    lax.fori_loop(0, NPAD // 16, zbody, ())

    ones16 = jnp.ones((16,), jnp.float32)
    row0 = jnp.zeros((16,), jnp.int32)
    row1 = jnp.ones((16,), jnp.int32)

    def body(i, _):
        s16 = sidx[pl.ds(i * 16, 16)]
        d16 = didx[pl.ds(i * 16, 16)]
        plsc.addupdate_scatter(odid, [row0, s16], ones16)
        plsc.addupdate_scatter(odid, [row1, d16], ones16)
        return ()

    lax.fori_loop(0, EPW // 16, body, ())
    pltpu.sync_copy(odid, out_hbm.at[wid])


@functools.partial(
    pl.kernel,
    out_type=jax.ShapeDtypeStruct((NW, CPT, NPAD), jnp.float32),
    mesh=_MESH,
    compiler_params=_SC_PARAMS,
    scratch_types=[
        pltpu.VMEM((CH,), jnp.int32),
        pltpu.VMEM((CH,), jnp.int32),
        pltpu.VMEM((CH,), jnp.int32),
        pltpu.VMEM((CH,), jnp.int32),
        pltpu.VMEM((CPT, NPAD), jnp.float32),
        pltpu.VMEM((CPT, NPAD), jnp.float32),
        pltpu.SemaphoreType.DMA,
        pltpu.SemaphoreType.DMA,
        pltpu.SemaphoreType.DMA,
        pltpu.SemaphoreType.DMA,
    ],
)
def _agg_kernel(ht_hbm, src_hbm, dst_hbm, out_hbm,
                sidx0, sidx1, didx0, didx1, tab, acc,
                sem_s0, sem_s1, sem_d0, sem_d1):
    c = lax.axis_index("c")
    s = lax.axis_index("s")
    wid = c * NS + s
    pltpu.sync_copy(ht_hbm.at[wid], tab)

    zero16 = jnp.zeros((16,), jnp.float32)

    def zbody(i, _):
        for j in range(CPT):
            acc[j, pl.ds(i * 16, 16)] = zero16
        return ()

    lax.fori_loop(0, NPAD // 16, zbody, ())

    jrows = [jnp.full((16,), j, jnp.int32) for j in range(CPT)]

    def start(k, sbuf, dbuf, ssem, dsem):
        pltpu.async_copy(src_hbm.at[pl.ds(k * CH, CH)], sbuf, ssem)
        pltpu.async_copy(dst_hbm.at[pl.ds(k * CH, CH)], dbuf, dsem)

    def wait(sbuf, dbuf, ssem, dsem):
        pltpu.make_async_copy(src_hbm.at[pl.ds(0, CH)], sbuf, ssem).wait()
        pltpu.make_async_copy(dst_hbm.at[pl.ds(0, CH)], dbuf, dsem).wait()

    def process(sbuf, dbuf):
        def body(i, _):
            for u in range(UNROLL):
                off = i * (16 * UNROLL) + u * 16
                s16 = sbuf[pl.ds(off, 16)]
                d16 = dbuf[pl.ds(off, 16)]
                for j in range(CPT):
                    vals = plsc.load_gather(tab, [jrows[j], s16])
                    plsc.addupdate_scatter(acc, [jrows[j], d16], vals)
            return ()

        lax.fori_loop(0, CH // (16 * UNROLL), body, ())

    start(0, sidx0, didx0, sem_s0, sem_d0)
    start(1, sidx1, didx1, sem_s1, sem_d1)

    def chunk2_body(kk, _):
        k = kk * 2
        wait(sidx0, didx0, sem_s0, sem_d0)
        process(sidx0, didx0)

        @pl.when(k + 2 < NCH)
        def _p0():
            start(k + 2, sidx0, didx0, sem_s0, sem_d0)

        wait(sidx1, didx1, sem_s1, sem_d1)
        process(sidx1, didx1)

        @pl.when(k + 3 < NCH)
        def _p1():
            start(k + 3, sidx1, didx1, sem_s1, sem_d1)

        return ()

    lax.fori_loop(0, NCH // 2, chunk2_body, ())
    pltpu.sync_copy(acc, out_hbm.at[wid])


_R = 1024


def _prep_body(ft_ref, deg_ref, o_ref):
    deg = jnp.sum(deg_ref[...], axis=0)
    ns = lax.rsqrt(jnp.maximum(deg[0:1, :], 1.0))
    o_ref[...] = ft_ref[...] * ns


def _tc_prep(ft, degP):
    return pl.pallas_call(
        _prep_body,
        out_shape=jax.ShapeDtypeStruct((D, NPAD), jnp.float32),
        grid=(NPAD // _R,),
        in_specs=[
            pl.BlockSpec((D, _R), lambda i: (0, i)),
            pl.BlockSpec((NW, 2, _R), lambda i: (0, 0, i)),
        ],
        out_specs=pl.BlockSpec((D, _R), lambda i: (0, i)),
    )(ft, degP)


def _dense_body(apply_src_norm, aggt_ref, deg_ref, w_ref, b_ref, o_ref):
    deg = jnp.sum(deg_ref[...], axis=0)
    nd = lax.rsqrt(jnp.maximum(deg[1:2, :], 1.0))
    x = aggt_ref[...] * nd
    y = lax.dot_general(w_ref[...], x, (((0,), (0,)), ((), ())),
                        preferred_element_type=jnp.float32)
    y = y + b_ref[:, 0:1]
    y = jnp.maximum(y, 0.0)
    if apply_src_norm:
        ns = lax.rsqrt(jnp.maximum(deg[0:1, :], 1.0))
        y = y * ns
    o_ref[...] = y


def _tc_dense(aggT, degP, W, b_bc, apply_src_norm):
    return pl.pallas_call(
        functools.partial(_dense_body, apply_src_norm),
        out_shape=jax.ShapeDtypeStruct((D, NPAD), jnp.float32),
        grid=(NPAD // _R,),
        in_specs=[
            pl.BlockSpec((D, _R), lambda i: (0, i)),
            pl.BlockSpec((NW, 2, _R), lambda i: (0, 0, i)),
            pl.BlockSpec((D, D), lambda i: (0, 0)),
            pl.BlockSpec((D, D), lambda i: (0, 0)),
        ],
        out_specs=pl.BlockSpec((D, _R), lambda i: (0, i)),
    )(aggT, degP, W, b_bc)


def kernel(features, edge_index, W1, b1, W2, b2):
    src = edge_index[0]
    dst = edge_index[1]

    degP = _deg_kernel(src, dst)

    ft = jnp.zeros((D, NPAD), jnp.float32).at[:, :N].set(features.T)
    hT0 = _tc_prep(ft, degP)

    b1_bc = jnp.broadcast_to(b1.reshape(D, 1), (D, D))
    b2_bc = jnp.broadcast_to(b2.reshape(D, 1), (D, D))

    agg1 = _agg_kernel(hT0.reshape(NW, CPT, NPAD), src, dst)
    h1T = _tc_dense(agg1.reshape(D, NPAD), degP, W1, b1_bc, True)

    agg2 = _agg_kernel(h1T.reshape(NW, CPT, NPAD), src, dst)
    o2T = _tc_dense(agg2.reshape(D, NPAD), degP, W2, b2_bc, False)
    return o2T[:, :N].T


# speedup vs baseline: 4.9078x; 1.4801x over previous
import functools

import jax
import jax.numpy as jnp
from jax import lax
from jax.experimental import pallas as pl
from jax.experimental.pallas import tpu as pltpu
from jax.experimental.pallas import tpu_sc as plsc

N = 10000
NPAD = 10240
E = 320000
D = 128

NC = 2
NS = 16
NW = NC * NS
EPW = E // NW
CPT = D // NW
CH = 6400
NCH = E // CH
UNROLL = 4

_MESH = plsc.VectorSubcoreMesh(core_axis_name="c", subcore_axis_name="s")
_SC_PARAMS = pltpu.CompilerParams(needs_layout_passes=False)


@functools.partial(
    pl.kernel,
    out_type=jax.ShapeDtypeStruct((NW, 2, NPAD), jnp.float32),
    mesh=_MESH,
    compiler_params=_SC_PARAMS,
    scratch_types=[
        pltpu.VMEM((EPW,), jnp.int32),
        pltpu.VMEM((EPW,), jnp.int32),
        pltpu.VMEM((2, NPAD), jnp.float32),
    ],
)
def _deg_kernel(src_hbm, dst_hbm, out_hbm, sidx, didx, odid):
    c = lax.axis_index("c")
    s = lax.axis_index("s")
    wid = c * NS + s
    base = wid * EPW
    pltpu.sync_copy(src_hbm.at[pl.ds(base, EPW)], sidx)
    pltpu.sync_copy(dst_hbm.at[pl.ds(base, EPW)], didx)

    zero16 = jnp.zeros((16,), jnp.float32)

    def zbody(i, _):
        odid[0, pl.ds(i * 16, 16)] = zero16
        odid[1, pl.ds(i * 16, 16)] = zero16
        return ()

    lax.fori_loop(0, NPAD // 16, zbody, ())

    ones16 = jnp.ones((16,), jnp.float32)
    row0 = jnp.zeros((16,), jnp.int32)
    row1 = jnp.ones((16,), jnp.int32)

    def body(i, _):
        s16 = sidx[pl.ds(i * 16, 16)]
        d16 = didx[pl.ds(i * 16, 16)]
        plsc.addupdate_scatter(odid, [row0, s16], ones16)
        plsc.addupdate_scatter(odid, [row1, d16], ones16)
        return ()

    lax.fori_loop(0, EPW // 16, body, ())
    pltpu.sync_copy(odid, out_hbm.at[wid])


@functools.partial(
    pl.kernel,
    out_type=jax.ShapeDtypeStruct((NW, NPAD * CPT), jnp.float32),
    mesh=_MESH,
    compiler_params=_SC_PARAMS,
    scratch_types=[
        pltpu.VMEM((CH,), jnp.int32),
        pltpu.VMEM((CH,), jnp.int32),
        pltpu.VMEM((CH,), jnp.int32),
        pltpu.VMEM((CH,), jnp.int32),
        pltpu.VMEM((NPAD * CPT,), jnp.float32),
        pltpu.VMEM((NPAD * CPT,), jnp.float32),
        pltpu.SemaphoreType.DMA,
        pltpu.SemaphoreType.DMA,
        pltpu.SemaphoreType.DMA,
        pltpu.SemaphoreType.DMA,
    ],
)
def _agg_kernel(ht_hbm, src_hbm, dst_hbm, out_hbm,
                sidx0, sidx1, didx0, didx1, tab, acc,
                sem_s0, sem_s1, sem_d0, sem_d1):
    c = lax.axis_index("c")
    s = lax.axis_index("s")
    wid = c * NS + s
    pltpu.sync_copy(ht_hbm.at[wid], tab)

    zero16 = jnp.zeros((16,), jnp.float32)

    def zbody(i, _):
        for j in range(CPT):
            acc[pl.ds(i * 64 + j * 16, 16)] = zero16
        return ()

    lax.fori_loop(0, NPAD * CPT // 64, zbody, ())

    def start(k, sbuf, dbuf, ssem, dsem):
        pltpu.async_copy(src_hbm.at[pl.ds(k * CH, CH)], sbuf, ssem)
        pltpu.async_copy(dst_hbm.at[pl.ds(k * CH, CH)], dbuf, dsem)

    def wait(sbuf, dbuf, ssem, dsem):
        pltpu.make_async_copy(src_hbm.at[pl.ds(0, CH)], sbuf, ssem).wait()
        pltpu.make_async_copy(dst_hbm.at[pl.ds(0, CH)], dbuf, dsem).wait()

    def process(sbuf, dbuf):
        def body(i, _):
            sa, da = [], []
            for u in range(UNROLL):
                off = i * (16 * UNROLL) + u * 16
                sa.append(sbuf[pl.ds(off, 16)] * CPT)
                da.append(dbuf[pl.ds(off, 16)] * CPT)
            vals = [plsc.load_gather(tab, [sa[u] + j])
                    for u in range(UNROLL) for j in range(CPT)]
            k = 0
            for u in range(UNROLL):
                for j in range(CPT):
                    plsc.addupdate_scatter(acc, [da[u] + j], vals[k])
                    k += 1
            return ()

        lax.fori_loop(0, CH // (16 * UNROLL), body, ())

    start(0, sidx0, didx0, sem_s0, sem_d0)
    start(1, sidx1, didx1, sem_s1, sem_d1)

    def chunk2_body(kk, _):
        k = kk * 2
        wait(sidx0, didx0, sem_s0, sem_d0)
        process(sidx0, didx0)

        @pl.when(k + 2 < NCH)
        def _p0():
            start(k + 2, sidx0, didx0, sem_s0, sem_d0)

        wait(sidx1, didx1, sem_s1, sem_d1)
        process(sidx1, didx1)

        @pl.when(k + 3 < NCH)
        def _p1():
            start(k + 3, sidx1, didx1, sem_s1, sem_d1)

        return ()

    lax.fori_loop(0, NCH // 2, chunk2_body, ())
    pltpu.sync_copy(acc, out_hbm.at[wid])


_R = 1024


def _prep_body(ft_ref, deg_ref, o_ref):
    deg = jnp.sum(deg_ref[...], axis=0)
    ns = lax.rsqrt(jnp.maximum(deg[0:1, :], 1.0))
    o_ref[...] = ft_ref[...] * ns


def _tc_prep(ft, degP):
    return pl.pallas_call(
        _prep_body,
        out_shape=jax.ShapeDtypeStruct((D, NPAD), jnp.float32),
        grid=(NPAD // _R,),
        in_specs=[
            pl.BlockSpec((D, _R), lambda i: (0, i)),
            pl.BlockSpec((NW, 2, _R), lambda i: (0, 0, i)),
        ],
        out_specs=pl.BlockSpec((D, _R), lambda i: (0, i)),
    )(ft, degP)


def _dense_body(apply_src_norm, aggt_ref, deg_ref, w_ref, b_ref, o_ref):
    deg = jnp.sum(deg_ref[...], axis=0)
    nd = lax.rsqrt(jnp.maximum(deg[1:2, :], 1.0))
    x = aggt_ref[...] * nd
    y = lax.dot_general(w_ref[...], x, (((0,), (0,)), ((), ())),
                        preferred_element_type=jnp.float32)
    y = y + b_ref[:, 0:1]
    y = jnp.maximum(y, 0.0)
    if apply_src_norm:
        ns = lax.rsqrt(jnp.maximum(deg[0:1, :], 1.0))
        y = y * ns
    o_ref[...] = y


def _tc_dense(aggT, degP, W, b_bc, apply_src_norm):
    return pl.pallas_call(
        functools.partial(_dense_body, apply_src_norm),
        out_shape=jax.ShapeDtypeStruct((D, NPAD), jnp.float32),
        grid=(NPAD // _R,),
        in_specs=[
            pl.BlockSpec((D, _R), lambda i: (0, i)),
            pl.BlockSpec((NW, 2, _R), lambda i: (0, 0, i)),
            pl.BlockSpec((D, D), lambda i: (0, 0)),
            pl.BlockSpec((D, D), lambda i: (0, 0)),
        ],
        out_specs=pl.BlockSpec((D, _R), lambda i: (0, i)),
    )(aggT, degP, W, b_bc)


def kernel(features, edge_index, W1, b1, W2, b2):
    src = edge_index[0]
    dst = edge_index[1]

    degP = _deg_kernel(src, dst)

    ft = jnp.zeros((D, NPAD), jnp.float32).at[:, :N].set(features.T)
    hT0 = _tc_prep(ft, degP)

    b1_bc = jnp.broadcast_to(b1.reshape(D, 1), (D, D))
    b2_bc = jnp.broadcast_to(b2.reshape(D, 1), (D, D))

    def to_sc(hT):
        return hT.reshape(NW, CPT, NPAD).transpose(0, 2, 1).reshape(NW, NPAD * CPT)

    def from_sc(agg):
        return agg.reshape(NW, NPAD, CPT).transpose(0, 2, 1).reshape(D, NPAD)

    agg1 = _agg_kernel(to_sc(hT0), src, dst)
    h1T = _tc_dense(from_sc(agg1), degP, W1, b1_bc, True)

    agg2 = _agg_kernel(to_sc(h1T), src, dst)
    o2T = _tc_dense(from_sc(agg2), degP, W2, b2_bc, False)
    return o2T[:, :N].T


# speedup vs baseline: 7.6421x; 1.5571x over previous
import functools

import jax
import jax.numpy as jnp
from jax import lax
from jax.experimental import pallas as pl
from jax.experimental.pallas import tpu as pltpu
from jax.experimental.pallas import tpu_sc as plsc

N = 10000
NPAD = 10240
E = 320000
D = 128

NC = 2
NS = 16
NW = NC * NS
EPW = E // NW
CPT = D // NW
CH = 6400
NCH = E // CH
UNROLL = 4

_MESH = plsc.VectorSubcoreMesh(core_axis_name="c", subcore_axis_name="s")
_SC_PARAMS = pltpu.CompilerParams(needs_layout_passes=False)


@functools.partial(
    pl.kernel,
    out_type=jax.ShapeDtypeStruct((NW, 2, NPAD), jnp.float32),
    mesh=_MESH,
    compiler_params=_SC_PARAMS,
    scratch_types=[
        pltpu.VMEM((EPW,), jnp.int32),
        pltpu.VMEM((EPW,), jnp.int32),
        pltpu.VMEM((2, NPAD), jnp.float32),
    ],
)
def _deg_kernel(src_hbm, dst_hbm, out_hbm, sidx, didx, odid):
    c = lax.axis_index("c")
    s = lax.axis_index("s")
    wid = c * NS + s
    base = wid * EPW
    pltpu.sync_copy(src_hbm.at[pl.ds(base, EPW)], sidx)
    pltpu.sync_copy(dst_hbm.at[pl.ds(base, EPW)], didx)

    zero16 = jnp.zeros((16,), jnp.float32)

    def zbody(i, _):
        odid[0, pl.ds(i * 16, 16)] = zero16
        odid[1, pl.ds(i * 16, 16)] = zero16
        return ()

    lax.fori_loop(0, NPAD // 16, zbody, ())

    ones16 = jnp.ones((16,), jnp.float32)
    row0 = jnp.zeros((16,), jnp.int32)
    row1 = jnp.ones((16,), jnp.int32)

    def body(i, _):
        s16 = sidx[pl.ds(i * 16, 16)]
        d16 = didx[pl.ds(i * 16, 16)]
        plsc.addupdate_scatter(odid, [row0, s16], ones16)
        plsc.addupdate_scatter(odid, [row1, d16], ones16)
        return ()

    lax.fori_loop(0, EPW // 16, body, ())
    pltpu.sync_copy(odid, out_hbm.at[wid])


@functools.partial(
    pl.kernel,
    out_type=jax.ShapeDtypeStruct((NW, CPT * NPAD), jnp.float32),
    mesh=_MESH,
    compiler_params=_SC_PARAMS,
    scratch_types=[
        pltpu.VMEM((CH,), jnp.int32),
        pltpu.VMEM((CH,), jnp.int32),
        pltpu.VMEM((CH,), jnp.int32),
        pltpu.VMEM((CH,), jnp.int32),
        pltpu.VMEM((CPT * NPAD,), jnp.float32),
        pltpu.VMEM((CPT * NPAD,), jnp.float32),
        pltpu.SemaphoreType.DMA,
        pltpu.SemaphoreType.DMA,
        pltpu.SemaphoreType.DMA,
        pltpu.SemaphoreType.DMA,
    ],
)
def _agg_kernel(ht_hbm, src_hbm, dst_hbm, out_hbm,
                sidx0, sidx1, didx0, didx1, tab, acc,
                sem_s0, sem_s1, sem_d0, sem_d1):
    c = lax.axis_index("c")
    s = lax.axis_index("s")
    wid = c * NS + s
    pltpu.sync_copy(ht_hbm.at[wid], tab)

    zero16 = jnp.zeros((16,), jnp.float32)

    def zbody(i, _):
        for j in range(CPT):
            acc[pl.ds(i * 64 + j * 16, 16)] = zero16
        return ()

    lax.fori_loop(0, CPT * NPAD // 64, zbody, ())

    def start(k, sbuf, dbuf, ssem, dsem):
        pltpu.async_copy(src_hbm.at[pl.ds(k * CH, CH)], sbuf, ssem)
        pltpu.async_copy(dst_hbm.at[pl.ds(k * CH, CH)], dbuf, dsem)

    def wait(sbuf, dbuf, ssem, dsem):
        pltpu.make_async_copy(src_hbm.at[pl.ds(0, CH)], sbuf, ssem).wait()
        pltpu.make_async_copy(dst_hbm.at[pl.ds(0, CH)], dbuf, dsem).wait()

    def process(sbuf, dbuf):
        def body(i, _):
            sa, da = [], []
            for u in range(UNROLL):
                off = i * (16 * UNROLL) + u * 16
                sa.append(sbuf[pl.ds(off, 16)])
                da.append(dbuf[pl.ds(off, 16)])
            vals = [plsc.load_gather(tab, [sa[u] + (j * NPAD)])
                    for u in range(UNROLL) for j in range(CPT)]
            k = 0
            for u in range(UNROLL):
                for j in range(CPT):
                    plsc.addupdate_scatter(acc, [da[u] + (j * NPAD)], vals[k])
                    k += 1
            return ()

        lax.fori_loop(0, CH // (16 * UNROLL), body, ())

    start(0, sidx0, didx0, sem_s0, sem_d0)
    start(1, sidx1, didx1, sem_s1, sem_d1)

    def chunk2_body(kk, _):
        k = kk * 2
        wait(sidx0, didx0, sem_s0, sem_d0)
        process(sidx0, didx0)

        @pl.when(k + 2 < NCH)
        def _p0():
            start(k + 2, sidx0, didx0, sem_s0, sem_d0)

        wait(sidx1, didx1, sem_s1, sem_d1)
        process(sidx1, didx1)

        @pl.when(k + 3 < NCH)
        def _p1():
            start(k + 3, sidx1, didx1, sem_s1, sem_d1)

        return ()

    lax.fori_loop(0, NCH // 2, chunk2_body, ())
    pltpu.sync_copy(acc, out_hbm.at[wid])


_R = 1024


def _prep_body(ft_ref, deg_ref, o_ref):
    deg = jnp.sum(deg_ref[...], axis=0)
    ns = lax.rsqrt(jnp.maximum(deg[0:1, :], 1.0))
    o_ref[...] = ft_ref[...] * ns


def _tc_prep(ft, degP):
    return pl.pallas_call(
        _prep_body,
        out_shape=jax.ShapeDtypeStruct((D, NPAD), jnp.float32),
        grid=(NPAD // _R,),
        in_specs=[
            pl.BlockSpec((D, _R), lambda i: (0, i)),
            pl.BlockSpec((NW, 2, _R), lambda i: (0, 0, i)),
        ],
        out_specs=pl.BlockSpec((D, _R), lambda i: (0, i)),
    )(ft, degP)


def _dense_body(apply_src_norm, aggt_ref, deg_ref, w_ref, b_ref, o_ref):
    deg = jnp.sum(deg_ref[...], axis=0)
    nd = lax.rsqrt(jnp.maximum(deg[1:2, :], 1.0))
    x = aggt_ref[...] * nd
    y = lax.dot_general(w_ref[...], x, (((0,), (0,)), ((), ())),
                        preferred_element_type=jnp.float32)
    y = y + b_ref[:, 0:1]
    y = jnp.maximum(y, 0.0)
    if apply_src_norm:
        ns = lax.rsqrt(jnp.maximum(deg[0:1, :], 1.0))
        y = y * ns
    o_ref[...] = y


def _tc_dense(aggT, degP, W, b_bc, apply_src_norm):
    return pl.pallas_call(
        functools.partial(_dense_body, apply_src_norm),
        out_shape=jax.ShapeDtypeStruct((D, NPAD), jnp.float32),
        grid=(NPAD // _R,),
        in_specs=[
            pl.BlockSpec((D, _R), lambda i: (0, i)),
            pl.BlockSpec((NW, 2, _R), lambda i: (0, 0, i)),
            pl.BlockSpec((D, D), lambda i: (0, 0)),
            pl.BlockSpec((D, D), lambda i: (0, 0)),
        ],
        out_specs=pl.BlockSpec((D, _R), lambda i: (0, i)),
    )(aggT, degP, W, b_bc)


def kernel(features, edge_index, W1, b1, W2, b2):
    src = edge_index[0]
    dst = edge_index[1]

    degP = _deg_kernel(src, dst)

    ft = jnp.zeros((D, NPAD), jnp.float32).at[:, :N].set(features.T)
    hT0 = _tc_prep(ft, degP)

    b1_bc = jnp.broadcast_to(b1.reshape(D, 1), (D, D))
    b2_bc = jnp.broadcast_to(b2.reshape(D, 1), (D, D))

    def to_sc(hT):
        return hT.reshape(NW, CPT * NPAD)

    def from_sc(agg):
        return agg.reshape(D, NPAD)

    agg1 = _agg_kernel(to_sc(hT0), src, dst)
    h1T = _tc_dense(from_sc(agg1), degP, W1, b1_bc, True)

    agg2 = _agg_kernel(to_sc(h1T), src, dst)
    o2T = _tc_dense(from_sc(agg2), degP, W2, b2_bc, False)
    return o2T[:, :N].T


# speedup vs baseline: 7.7109x; 1.0090x over previous
import functools

import jax
import jax.numpy as jnp
from jax import lax
from jax.experimental import pallas as pl
from jax.experimental.pallas import tpu as pltpu
from jax.experimental.pallas import tpu_sc as plsc

N = 10000
NPAD = 10240
E = 320000
D = 128

NC = 2
NS = 16
NW = NC * NS
EPW = E // NW
CPT = D // NW
CH = 6400
NCH = E // CH
UNROLL = 8

_MESH = plsc.VectorSubcoreMesh(core_axis_name="c", subcore_axis_name="s")
_SC_PARAMS = pltpu.CompilerParams(needs_layout_passes=False)


@functools.partial(
    pl.kernel,
    out_type=jax.ShapeDtypeStruct((NW, 2, NPAD), jnp.float32),
    mesh=_MESH,
    compiler_params=_SC_PARAMS,
    scratch_types=[
        pltpu.VMEM((EPW,), jnp.int32),
        pltpu.VMEM((EPW,), jnp.int32),
        pltpu.VMEM((2, NPAD), jnp.float32),
    ],
)
def _deg_kernel(src_hbm, dst_hbm, out_hbm, sidx, didx, odid):
    c = lax.axis_index("c")
    s = lax.axis_index("s")
    wid = c * NS + s
    base = wid * EPW
    pltpu.sync_copy(src_hbm.at[pl.ds(base, EPW)], sidx)
    pltpu.sync_copy(dst_hbm.at[pl.ds(base, EPW)], didx)

    zero16 = jnp.zeros((16,), jnp.float32)

    def zbody(i, _):
        odid[0, pl.ds(i * 16, 16)] = zero16
        odid[1, pl.ds(i * 16, 16)] = zero16
        return ()

    lax.fori_loop(0, NPAD // 16, zbody, ())

    ones16 = jnp.ones((16,), jnp.float32)
    row0 = jnp.zeros((16,), jnp.int32)
    row1 = jnp.ones((16,), jnp.int32)

    def body(i, _):
        s16 = sidx[pl.ds(i * 16, 16)]
        d16 = didx[pl.ds(i * 16, 16)]
        plsc.addupdate_scatter(odid, [row0, s16], ones16)
        plsc.addupdate_scatter(odid, [row1, d16], ones16)
        return ()

    lax.fori_loop(0, EPW // 16, body, ())
    pltpu.sync_copy(odid, out_hbm.at[wid])


@functools.partial(
    pl.kernel,
    out_type=jax.ShapeDtypeStruct((NW, CPT * NPAD), jnp.float32),
    mesh=_MESH,
    compiler_params=_SC_PARAMS,
    scratch_types=[
        pltpu.VMEM((CH,), jnp.int32),
        pltpu.VMEM((CH,), jnp.int32),
        pltpu.VMEM((CH,), jnp.int32),
        pltpu.VMEM((CH,), jnp.int32),
        pltpu.VMEM((CPT * NPAD,), jnp.float32),
        pltpu.VMEM((CPT * NPAD,), jnp.float32),
        pltpu.SemaphoreType.DMA,
        pltpu.SemaphoreType.DMA,
        pltpu.SemaphoreType.DMA,
        pltpu.SemaphoreType.DMA,
    ],
)
def _agg_kernel(ht_hbm, src_hbm, dst_hbm, out_hbm,
                sidx0, sidx1, didx0, didx1, tab, acc,
                sem_s0, sem_s1, sem_d0, sem_d1):
    c = lax.axis_index("c")
    s = lax.axis_index("s")
    wid = c * NS + s
    pltpu.sync_copy(ht_hbm.at[wid], tab)

    zero16 = jnp.zeros((16,), jnp.float32)

    def zbody(i, _):
        for j in range(CPT):
            acc[pl.ds(i * 64 + j * 16, 16)] = zero16
        return ()

    lax.fori_loop(0, CPT * NPAD // 64, zbody, ())

    def start(k, sbuf, dbuf, ssem, dsem):
        pltpu.async_copy(src_hbm.at[pl.ds(k * CH, CH)], sbuf, ssem)
        pltpu.async_copy(dst_hbm.at[pl.ds(k * CH, CH)], dbuf, dsem)

    def wait(sbuf, dbuf, ssem, dsem):
        pltpu.make_async_copy(src_hbm.at[pl.ds(0, CH)], sbuf, ssem).wait()
        pltpu.make_async_copy(dst_hbm.at[pl.ds(0, CH)], dbuf, dsem).wait()

    def process(sbuf, dbuf):
        def body(i, _):
            sa, da = [], []
            for u in range(UNROLL):
                off = i * (16 * UNROLL) + u * 16
                sa.append(sbuf[pl.ds(off, 16)])
                da.append(dbuf[pl.ds(off, 16)])
            vals = [plsc.load_gather(tab, [sa[u] + (j * NPAD)])
                    for u in range(UNROLL) for j in range(CPT)]
            k = 0
            for u in range(UNROLL):
                for j in range(CPT):
                    plsc.addupdate_scatter(acc, [da[u] + (j * NPAD)], vals[k])
                    k += 1
            return ()

        lax.fori_loop(0, CH // (16 * UNROLL), body, ())

    start(0, sidx0, didx0, sem_s0, sem_d0)
    start(1, sidx1, didx1, sem_s1, sem_d1)

    def chunk2_body(kk, _):
        k = kk * 2
        wait(sidx0, didx0, sem_s0, sem_d0)
        process(sidx0, didx0)

        @pl.when(k + 2 < NCH)
        def _p0():
            start(k + 2, sidx0, didx0, sem_s0, sem_d0)

        wait(sidx1, didx1, sem_s1, sem_d1)
        process(sidx1, didx1)

        @pl.when(k + 3 < NCH)
        def _p1():
            start(k + 3, sidx1, didx1, sem_s1, sem_d1)

        return ()

    lax.fori_loop(0, NCH // 2, chunk2_body, ())
    pltpu.sync_copy(acc, out_hbm.at[wid])


_R = 1024


def _prep_body(ft_ref, deg_ref, o_ref):
    deg = jnp.sum(deg_ref[...], axis=0)
    ns = lax.rsqrt(jnp.maximum(deg[0:1, :], 1.0))
    o_ref[...] = ft_ref[...] * ns


def _tc_prep(ft, degP):
    return pl.pallas_call(
        _prep_body,
        out_shape=jax.ShapeDtypeStruct((D, NPAD), jnp.float32),
        grid=(NPAD // _R,),
        in_specs=[
            pl.BlockSpec((D, _R), lambda i: (0, i)),
            pl.BlockSpec((NW, 2, _R), lambda i: (0, 0, i)),
        ],
        out_specs=pl.BlockSpec((D, _R), lambda i: (0, i)),
    )(ft, degP)


def _dense_body(apply_src_norm, aggt_ref, deg_ref, w_ref, b_ref, o_ref):
    deg = jnp.sum(deg_ref[...], axis=0)
    nd = lax.rsqrt(jnp.maximum(deg[1:2, :], 1.0))
    x = aggt_ref[...] * nd
    y = lax.dot_general(w_ref[...], x, (((0,), (0,)), ((), ())),
                        preferred_element_type=jnp.float32)
    y = y + b_ref[:, 0:1]
    y = jnp.maximum(y, 0.0)
    if apply_src_norm:
        ns = lax.rsqrt(jnp.maximum(deg[0:1, :], 1.0))
        y = y * ns
    o_ref[...] = y


def _tc_dense(aggT, degP, W, b_bc, apply_src_norm):
    return pl.pallas_call(
        functools.partial(_dense_body, apply_src_norm),
        out_shape=jax.ShapeDtypeStruct((D, NPAD), jnp.float32),
        grid=(NPAD // _R,),
        in_specs=[
            pl.BlockSpec((D, _R), lambda i: (0, i)),
            pl.BlockSpec((NW, 2, _R), lambda i: (0, 0, i)),
            pl.BlockSpec((D, D), lambda i: (0, 0)),
            pl.BlockSpec((D, D), lambda i: (0, 0)),
        ],
        out_specs=pl.BlockSpec((D, _R), lambda i: (0, i)),
    )(aggT, degP, W, b_bc)


def kernel(features, edge_index, W1, b1, W2, b2):
    src = edge_index[0]
    dst = edge_index[1]

    degP = _deg_kernel(src, dst)

    ft = jnp.zeros((D, NPAD), jnp.float32).at[:, :N].set(features.T)
    hT0 = _tc_prep(ft, degP)

    b1_bc = jnp.broadcast_to(b1.reshape(D, 1), (D, D))
    b2_bc = jnp.broadcast_to(b2.reshape(D, 1), (D, D))

    def to_sc(hT):
        return hT.reshape(NW, CPT * NPAD)

    def from_sc(agg):
        return agg.reshape(D, NPAD)

    agg1 = _agg_kernel(to_sc(hT0), src, dst)
    h1T = _tc_dense(from_sc(agg1), degP, W1, b1_bc, True)

    agg2 = _agg_kernel(to_sc(h1T), src, dst)
    o2T = _tc_dense(from_sc(agg2), degP, W2, b2_bc, False)
    return o2T[:, :N].T
